# baseline (speedup 1.0000x reference)
import functools

import jax
import jax.numpy as jnp
import numpy as np
from jax import lax
from jax.experimental import pallas as pl
from jax.experimental.pallas import tpu as pltpu
from jax.experimental.pallas import tpu_sc as plsc

try:
    from jax.experimental import enable_x64 as _enable_x64
except ImportError:
    from jax._src.config import enable_x64 as _enable_x64

N_NEIGH = 16
N_SITES = 40
N_VERTICES = 512
NW = 32
VPW = N_VERTICES // NW

F32 = jnp.float32
I32 = jnp.int32

_BIG = np.float32(1e30)
_EPS = 2.0 ** -24
_MREL = np.float32(16.0 * _EPS)
_MCMP = np.float32(16.0 * _EPS)


def _two_sum(a, b):
    s = a + b
    bb = s - a
    return s, (a - (s - bb)) + (b - bb)


def _split(a):
    c = F32(4097.0) * a
    ah = c - (c - a)
    return ah, a - ah


def _two_prod(a, b):
    p = a * b
    ah, al = _split(a)
    bh, bl = _split(b)
    e = ((ah * bh - p) + ah * bl + al * bh) + al * bl
    return p, e


def _df_add(a, b):
    s, e = _two_sum(a[0], b[0])
    e = e + (a[1] + b[1])
    return _two_sum(s, e)


def _df_sub(a, b):
    return _df_add(a, (-b[0], -b[1]))


def _df_mul(a, b):
    p, e = _two_prod(a[0], b[0])
    e = e + (a[0] * b[1] + a[1] * b[0])
    return _two_sum(p, e)


def _df_sq(a):
    p, e = _two_prod(a[0], a[0])
    e = e + F32(2.0) * (a[0] * a[1])
    return _two_sum(p, e)


def _df_diff(a, b):
    return _two_sum(a, -b)


def _df_lt(a, b):
    return (a[0] < b[0]) | ((a[0] == b[0]) & (a[1] < b[1]))


def _df_le(a, b):
    return (a[0] < b[0]) | ((a[0] == b[0]) & (a[1] <= b[1]))


def _df_pos(a):
    return (a[0] > 0) | ((a[0] == 0) & (a[1] > 0))


def _df_nonneg(a):
    return (a[0] > 0) | ((a[0] == 0) & (a[1] >= 0))


def _rsqrt_f32(h):
    i = lax.bitcast_convert_type(h, I32)
    g = lax.bitcast_convert_type(jnp.int32(0x5F3759DF) - (i >> 1), F32)
    for _ in range(3):
        hg = h * g
        g = g * (F32(1.5) - F32(0.5) * hg * g)
    return g


def _df_sqrt(x):
    h = x[0]
    g = _rsqrt_f32(h)
    s0 = h * g
    p, pe = _two_prod(s0, s0)
    t, te = _two_sum(h, -p)
    te = te + (x[1] - pe)
    corr = (t + te) * (F32(0.5) * g)
    return _two_sum(s0, corr)


def _bary_body(px_hbm, py_hbm, tx_hbm, ty_hbm, w_hbm, i_hbm,
               px_v, py_v, tx_v, ty_v,
               d2h_s, d2l_s, sh_s, sl_s,
               vxh_s, vxl_s, vyh_s, vyl_s,
               d00h_s, d00l_s, d02h_s, d02l_s,
               ow_v, oi_v):
    wid = lax.axis_index("s") * 2 + lax.axis_index("c")
    pltpu.sync_copy(px_hbm.at[wid], px_v)
    pltpu.sync_copy(py_hbm.at[wid], py_v)
    pltpu.sync_copy(tx_hbm, tx_v)
    pltpu.sync_copy(ty_hbm, ty_v)

    lanes = lax.iota(I32, VPW)
    inf = jnp.full((VPW,), jnp.inf, F32)
    fz = jnp.zeros((VPW,), F32)
    iz = jnp.zeros((VPW,), I32)
    bigv = jnp.full((VPW,), _BIG, F32)
    fals = jnp.zeros((VPW,), jnp.bool_)

    def store_rows(j, novalid, cidx, bu, bv, swap, w0, wu, wv):
        row = j * 3
        ow_v[row] = jnp.where(novalid, fz, w0)
        ow_v[row + 1] = jnp.where(novalid, fz, jnp.where(swap, wv, wu))
        ow_v[row + 2] = jnp.where(novalid, fz, jnp.where(swap, wu, wv))
        oi_v[row] = jnp.where(novalid, iz, cidx)
        oi_v[row + 1] = jnp.where(novalid, iz, jnp.where(swap, bv, bu))
        oi_v[row + 2] = jnp.where(novalid, iz, jnp.where(swap, bu, bv))

    def site_slow(j, tx, ty):

        def dist_k(k, carry):
            mh, ml, cidx = carry
            dx = _df_diff(tx, px_v[k])
            dy = _df_diff(ty, py_v[k])
            d2 = _df_add(_df_sq(dx), _df_sq(dy))
            d2h_s[k] = d2[0]
            d2l_s[k] = d2[1]
            s = _df_sqrt(d2)
            sh_s[k] = s[0]
            sl_s[k] = s[1]
            less = _df_lt(d2, (mh, ml))
            return (jnp.where(less, d2[0], mh),
                    jnp.where(less, d2[1], ml),
                    jnp.where(less, k, cidx))

        _, _, cidx = lax.fori_loop(np.int32(0), np.int32(N_NEIGH), dist_k, (inf, fz, iz))

        pcx = plsc.load_gather(px_v, [cidx, lanes])
        pcy = plsc.load_gather(py_v, [cidx, lanes])
        v2x = _df_diff(tx, pcx)
        v2y = _df_diff(ty, pcy)

        def dots_k(k, _):
            vx = _df_diff(px_v[k], pcx)
            vy = _df_diff(py_v[k], pcy)
            d00 = _df_add(_df_sq(vx), _df_sq(vy))
            d02 = _df_add(_df_mul(vx, v2x), _df_mul(vy, v2y))
            vxh_s[k] = vx[0]
            vxl_s[k] = vx[1]
            vyh_s[k] = vy[0]
            vyl_s[k] = vy[1]
            d00h_s[k] = d00[0]
            d00l_s[k] = d00[1]
            d02h_s[k] = d02[0]
            d02l_s[k] = d02[1]
            return jnp.int32(0)

        lax.fori_loop(np.int32(0), np.int32(N_NEIGH), dots_k, jnp.int32(0))

        def pair_u(u, best):
            vxu = (vxh_s[u], vxl_s[u])
            vyu = (vyh_s[u], vyl_s[u])
            d00u = (d00h_s[u], d00l_s[u])
            d02u = (d02h_s[u], d02l_s[u])
            su = (sh_s[u], sl_s[u])
            u_ok = u != cidx

            def pair_v(v, best):
                (bch, bcl, bu, bv, bnuh, bnul, bnvh, bnvl, bdh, bdl) = best
                vxv = (vxh_s[v], vxl_s[v])
                vyv = (vyh_s[v], vyl_s[v])
                d00v = (d00h_s[v], d00l_s[v])
                d02v = (d02h_s[v], d02l_s[v])
                sv = (sh_s[v], sl_s[v])
                dot01 = _df_add(_df_mul(vxu, vxv), _df_mul(vyu, vyv))
                den = _df_sub(_df_mul(d00u, d00v), _df_sq(dot01))
                nu = _df_sub(_df_mul(d00v, d02u), _df_mul(dot01, d02v))
                nv = _df_sub(_df_mul(d00u, d02v), _df_mul(dot01, d02u))
                nsum = _df_add(nu, nv)
                valid = (_df_pos(den) & _df_nonneg(nu) & _df_nonneg(nv)
                         & _df_le(nsum, den) & u_ok & (v != cidx))
                cost = _df_add(su, sv)
                take = valid & _df_lt(cost, (bch, bcl))
                return (jnp.where(take, cost[0], bch),
                        jnp.where(take, cost[1], bcl),
                        jnp.where(take, u, bu),
                        jnp.where(take, v, bv),
                        jnp.where(take, nu[0], bnuh),
                        jnp.where(take, nu[1], bnul),
                        jnp.where(take, nv[0], bnvh),
                        jnp.where(take, nv[1], bnvl),
                        jnp.where(take, den[0], bdh),
                        jnp.where(take, den[1], bdl))

            return lax.fori_loop(u + jnp.int32(1), jnp.int32(N_NEIGH), pair_v, best)

        best0 = (inf, fz, iz, iz, fz, fz, fz, fz,
                 jnp.ones((VPW,), F32), fz)
        (bch, _, bu, bv, bnuh, bnul, bnvh, bnvl, bdh, bdl) = (
            lax.fori_loop(np.int32(0), np.int32(N_NEIGH), pair_u, best0))

        novalid = bch == inf
        dsum = bdh + bdl
        dsum = jnp.where(novalid, jnp.ones((VPW,), F32), dsum)
        wu = (bnuh + bnul) / dsum
        wv = (bnvh + bnvl) / dsum
        w0 = F32(1.0) - wu - wv
        d2u = (plsc.load_gather(d2h_s, [bu, lanes]),
               plsc.load_gather(d2l_s, [bu, lanes]))
        d2v = (plsc.load_gather(d2h_s, [bv, lanes]),
               plsc.load_gather(d2l_s, [bv, lanes]))
        swap = _df_lt(d2v, d2u)
        store_rows(j, novalid, cidx, bu, bv, swap, w0, wu, wv)

    def site(j, _):
        tx = tx_v[j]
        ty = ty_v[j]

        def fdist_k(k, carry):
            mf, cidx, amb = carry
            dx = tx - px_v[k]
            dy = ty - py_v[k]
            d2 = dx * dx + dy * dy
            d2h_s[k] = d2
            sh_s[k] = d2 * _rsqrt_f32(d2)
            amb = amb | (jnp.abs(d2 - mf) <= _MCMP * (d2 + mf))
            less = d2 < mf
            return (jnp.where(less, d2, mf),
                    jnp.where(less, k, cidx),
                    amb)

        _, cidxf, amb = plsc.parallel_loop(
            np.int32(0), np.int32(N_NEIGH),
            carry=(bigv, iz, fals))(lambda k, c: fdist_k(k, c))

        pcx = plsc.load_gather(px_v, [cidxf, lanes])
        pcy = plsc.load_gather(py_v, [cidxf, lanes])
        v2xf = tx - pcx
        v2yf = ty - pcy

        def fdots_k(k, _):
            vx = px_v[k] - pcx
            vy = py_v[k] - pcy
            vxh_s[k] = vx
            vyh_s[k] = vy
            d00h_s[k] = vx * vx + vy * vy
            t1 = vx * v2xf
            t2 = vy * v2yf
            d02h_s[k] = t1 + t2
            d02l_s[k] = jnp.abs(t1) + jnp.abs(t2)
            return jnp.int32(0)

        plsc.parallel_loop(
            np.int32(0), np.int32(N_NEIGH),
            carry=jnp.int32(0))(lambda k, c: fdots_k(k, c))

        def fpair_u(u, best):
            vxu = vxh_s[u]
            vyu = vyh_s[u]
            d00u = d00h_s[u]
            d02u = d02h_s[u]
            d02bu = d02l_s[u]
            su = sh_s[u]
            u_ok = u != cidxf

            def fpair_v(v, best):
                bc, bu, bv, amb, mac = best
                vxv = vxh_s[v]
                vyv = vyh_s[v]
                d00v = d00h_s[v]
                d02v = d02h_s[v]
                d02bv = d02l_s[v]
                sv = sh_s[v]
                t1 = vxu * vxv
                t2 = vyu * vyv
                q = t1 + t2
                qb = jnp.abs(t1) + jnp.abs(t2)
                p1 = d00u * d00v
                p2 = q * q
                den = p1 - p2
                bb = p1 + qb * qb
                a1 = d00v * d02u
                a2 = q * d02v
                nu = a1 - a2
                b1 = d00u * d02v
                b2 = q * d02u
                nv = b1 - b2
                bnu = d00v * d02bu + qb * d02bv
                bnv = d00u * d02bv + qb * d02bu
                nsum = nu + nv
                rem = den - nsum
                brem = bb + bnu + bnv
                pmask = u_ok & (v != cidxf)
                mden = _MREL * bb
                mnu = _MREL * bnu
                mnv = _MREL * bnv
                mrem = _MREL * brem
                dvalid = ((den > mden) & (nu > mnu)
                          & (nv > mnv) & (rem > mrem) & pmask)
                dinv = ((den < -mden) | (nu < -mnu)
                        | (nv < -mnv) | (rem < -mrem)
                        | ~pmask)
                pamb = ~dvalid & ~dinv
                cost = su + sv
                mc = _MCMP * (cost + bc)
                amb = amb | (dvalid & (jnp.abs(cost - bc) <= mc))
                mac = jnp.where(pamb & (cost < mac), cost, mac)
                take = dvalid & (cost < bc)
                return (jnp.where(take, cost, bc),
                        jnp.where(take, u, bu),
                        jnp.where(take, v, bv),
                        amb, mac)

            return plsc.parallel_loop(
                np.int32(u + 1), np.int32(N_NEIGH),
                unroll=2, carry=best)(lambda v, b: fpair_v(v, b))

        best = (bigv, iz, iz, amb, bigv)
        for _u in range(N_NEIGH - 1):
            best = fpair_u(_u, best)
        bc, bu, bv, amb, mac = best

        amb = amb | ((mac < bigv) & (mac < bc + _MCMP * (bc + mac)))
        novalid = bc == bigv
        d2uf = plsc.load_gather(d2h_s, [bu, lanes])
        d2vf = plsc.load_gather(d2h_s, [bv, lanes])
        amb = amb | (~novalid & (jnp.abs(d2vf - d2uf)
                                 <= _MCMP * (d2uf + d2vf)))
        swap = d2vf < d2uf

        pxu = plsc.load_gather(px_v, [bu, lanes])
        pyu = plsc.load_gather(py_v, [bu, lanes])
        pxv = plsc.load_gather(px_v, [bv, lanes])
        pyv = plsc.load_gather(py_v, [bv, lanes])
        vxu = _df_diff(pxu, pcx)
        vyu = _df_diff(pyu, pcy)
        vxv = _df_diff(pxv, pcx)
        vyv = _df_diff(pyv, pcy)
        v2x = _df_diff(tx, pcx)
        v2y = _df_diff(ty, pcy)
        d00u = _df_add(_df_sq(vxu), _df_sq(vyu))
        d00v = _df_add(_df_sq(vxv), _df_sq(vyv))
        d02u = _df_add(_df_mul(vxu, v2x), _df_mul(vyu, v2y))
        d02v = _df_add(_df_mul(vxv, v2x), _df_mul(vyv, v2y))
        dot01 = _df_add(_df_mul(vxu, vxv), _df_mul(vyu, vyv))
        den = _df_sub(_df_mul(d00u, d00v), _df_sq(dot01))
        nu = _df_sub(_df_mul(d00v, d02u), _df_mul(dot01, d02v))
        nv = _df_sub(_df_mul(d00u, d02v), _df_mul(dot01, d02u))
        dh = den[0] + den[1]
        dh = jnp.where(novalid | (dh == fz), jnp.ones((VPW,), F32), dh)
        wu = (nu[0] + nu[1]) / dh
        wv = (nv[0] + nv[1]) / dh
        w0 = F32(1.0) - wu - wv
        store_rows(j, novalid, cidxf, bu, bv, swap, w0, wu, wv)

        any_amb = jnp.max(jnp.where(amb, jnp.ones((VPW,), F32), fz)) > F32(0.0)

        @pl.when(any_amb)
        def _():
            site_slow(j, tx, ty)

        return jnp.int32(0)

    lax.fori_loop(np.int32(0), np.int32(N_SITES), site, jnp.int32(0))
    pltpu.sync_copy(ow_v, w_hbm.at[wid])
    pltpu.sync_copy(oi_v, i_hbm.at[wid])


_bary_sc = functools.partial(
    pl.kernel,
    out_type=[
        jax.ShapeDtypeStruct((NW, N_SITES * 3, VPW), F32),
        jax.ShapeDtypeStruct((NW, N_SITES * 3, VPW), I32),
    ],
    mesh=plsc.VectorSubcoreMesh(
        core_axis_name="c", subcore_axis_name="s",
        num_cores=2, num_subcores=16),
    compiler_params=pltpu.CompilerParams(needs_layout_passes=False),
    scratch_types=(
        [pltpu.VMEM((N_NEIGH, VPW), F32)] * 2
        + [pltpu.VMEM((N_SITES, VPW), F32)] * 2
        + [pltpu.VMEM((N_NEIGH, VPW), F32)] * 12
        + [pltpu.VMEM((N_SITES * 3, VPW), F32),
           pltpu.VMEM((N_SITES * 3, VPW), I32)]
    ),
)(_bary_body)


def kernel(template, projections):
    t = template.astype(F32).reshape(N_SITES, 2)
    p = projections.astype(F32)
    px = p[..., 0].reshape(NW, VPW, N_NEIGH).transpose(0, 2, 1)
    py = p[..., 1].reshape(NW, VPW, N_NEIGH).transpose(0, 2, 1)
    txb = jnp.broadcast_to(t[:, 0][:, None], (N_SITES, VPW)) + F32(0.0)
    tyb = jnp.broadcast_to(t[:, 1][:, None], (N_SITES, VPW)) + F32(0.0)
    with _enable_x64(False):
        w, idx = _bary_sc(px, py, txb, tyb)
    w = w.reshape(NW, N_SITES, 3, VPW).transpose(0, 3, 1, 2)
    w = w.reshape(N_VERTICES, 5, 8, 3)
    idx = idx.reshape(NW, N_SITES, 3, VPW).transpose(0, 3, 1, 2)
    idx = idx.reshape(N_VERTICES, 5, 8, 3)
    return w.astype(jnp.float64), idx.astype(jnp.int64)


# speedup vs baseline: 1.4562x; 1.4562x over previous
import functools

import jax
import jax.numpy as jnp
import numpy as np
from jax import lax
from jax.experimental import pallas as pl
from jax.experimental.pallas import tpu as pltpu
from jax.experimental.pallas import tpu_sc as plsc

try:
    from jax.experimental import enable_x64 as _enable_x64
except ImportError:
    from jax._src.config import enable_x64 as _enable_x64

N_NEIGH = 16
N_SITES = 40
N_VERTICES = 512
NW = 32
VPW = N_VERTICES // NW

F32 = jnp.float32
I32 = jnp.int32

_BIG = np.float32(1e30)
_EPS = 2.0 ** -24
_MREL = np.float32(16.0 * _EPS)
_MCMP = np.float32(16.0 * _EPS)


def _two_sum(a, b):
    s = a + b
    bb = s - a
    return s, (a - (s - bb)) + (b - bb)


def _split(a):
    c = F32(4097.0) * a
    ah = c - (c - a)
    return ah, a - ah


def _two_prod(a, b):
    p = a * b
    ah, al = _split(a)
    bh, bl = _split(b)
    e = ((ah * bh - p) + ah * bl + al * bh) + al * bl
    return p, e


def _df_add(a, b):
    s, e = _two_sum(a[0], b[0])
    e = e + (a[1] + b[1])
    return _two_sum(s, e)


def _df_sub(a, b):
    return _df_add(a, (-b[0], -b[1]))


def _df_mul(a, b):
    p, e = _two_prod(a[0], b[0])
    e = e + (a[0] * b[1] + a[1] * b[0])
    return _two_sum(p, e)


def _df_sq(a):
    p, e = _two_prod(a[0], a[0])
    e = e + F32(2.0) * (a[0] * a[1])
    return _two_sum(p, e)


def _df_diff(a, b):
    return _two_sum(a, -b)


def _df_lt(a, b):
    return (a[0] < b[0]) | ((a[0] == b[0]) & (a[1] < b[1]))


def _df_le(a, b):
    return (a[0] < b[0]) | ((a[0] == b[0]) & (a[1] <= b[1]))


def _df_pos(a):
    return (a[0] > 0) | ((a[0] == 0) & (a[1] > 0))


def _df_nonneg(a):
    return (a[0] > 0) | ((a[0] == 0) & (a[1] >= 0))


def _rsqrt_f32(h):
    i = lax.bitcast_convert_type(h, I32)
    g = lax.bitcast_convert_type(jnp.int32(0x5F3759DF) - (i >> 1), F32)
    for _ in range(3):
        hg = h * g
        g = g * (F32(1.5) - F32(0.5) * hg * g)
    return g


def _df_sqrt(x):
    h = x[0]
    g = _rsqrt_f32(h)
    s0 = h * g
    p, pe = _two_prod(s0, s0)
    t, te = _two_sum(h, -p)
    te = te + (x[1] - pe)
    corr = (t + te) * (F32(0.5) * g)
    return _two_sum(s0, corr)


def _bary_body(px_hbm, py_hbm, tx_hbm, ty_hbm, w_hbm, i_hbm,
               px_v, py_v, tx_v, ty_v,
               d2h_s, d2l_s, sh_s, sl_s,
               vxh_s, vxl_s, vyh_s, vyl_s,
               d00h_s, d00l_s, d02h_s, d02l_s,
               ow_v, oi_v):
    wid = lax.axis_index("s") * 2 + lax.axis_index("c")
    pltpu.sync_copy(px_hbm.at[wid], px_v)
    pltpu.sync_copy(py_hbm.at[wid], py_v)
    pltpu.sync_copy(tx_hbm, tx_v)
    pltpu.sync_copy(ty_hbm, ty_v)

    lanes = lax.iota(I32, VPW)
    inf = jnp.full((VPW,), jnp.inf, F32)
    fz = jnp.zeros((VPW,), F32)
    iz = jnp.zeros((VPW,), I32)
    bigv = jnp.full((VPW,), _BIG, F32)
    fals = jnp.zeros((VPW,), jnp.bool_)

    def store_rows(j, novalid, cidx, bu, bv, swap, w0, wu, wv):
        row = j * 3
        ow_v[row] = jnp.where(novalid, fz, w0)
        ow_v[row + 1] = jnp.where(novalid, fz, jnp.where(swap, wv, wu))
        ow_v[row + 2] = jnp.where(novalid, fz, jnp.where(swap, wu, wv))
        oi_v[row] = jnp.where(novalid, iz, cidx)
        oi_v[row + 1] = jnp.where(novalid, iz, jnp.where(swap, bv, bu))
        oi_v[row + 2] = jnp.where(novalid, iz, jnp.where(swap, bu, bv))

    def site_slow(j, tx, ty):

        def dist_k(k, carry):
            mh, ml, cidx = carry
            dx = _df_diff(tx, px_v[k])
            dy = _df_diff(ty, py_v[k])
            d2 = _df_add(_df_sq(dx), _df_sq(dy))
            d2h_s[k] = d2[0]
            d2l_s[k] = d2[1]
            s = _df_sqrt(d2)
            sh_s[k] = s[0]
            sl_s[k] = s[1]
            less = _df_lt(d2, (mh, ml))
            return (jnp.where(less, d2[0], mh),
                    jnp.where(less, d2[1], ml),
                    jnp.where(less, k, cidx))

        _, _, cidx = lax.fori_loop(np.int32(0), np.int32(N_NEIGH), dist_k, (inf, fz, iz))

        pcx = plsc.load_gather(px_v, [cidx, lanes])
        pcy = plsc.load_gather(py_v, [cidx, lanes])
        v2x = _df_diff(tx, pcx)
        v2y = _df_diff(ty, pcy)

        def dots_k(k, _):
            vx = _df_diff(px_v[k], pcx)
            vy = _df_diff(py_v[k], pcy)
            d00 = _df_add(_df_sq(vx), _df_sq(vy))
            d02 = _df_add(_df_mul(vx, v2x), _df_mul(vy, v2y))
            vxh_s[k] = vx[0]
            vxl_s[k] = vx[1]
            vyh_s[k] = vy[0]
            vyl_s[k] = vy[1]
            d00h_s[k] = d00[0]
            d00l_s[k] = d00[1]
            d02h_s[k] = d02[0]
            d02l_s[k] = d02[1]
            return jnp.int32(0)

        lax.fori_loop(np.int32(0), np.int32(N_NEIGH), dots_k, jnp.int32(0))

        def pair_u(u, best):
            vxu = (vxh_s[u], vxl_s[u])
            vyu = (vyh_s[u], vyl_s[u])
            d00u = (d00h_s[u], d00l_s[u])
            d02u = (d02h_s[u], d02l_s[u])
            su = (sh_s[u], sl_s[u])
            u_ok = u != cidx

            def pair_v(v, best):
                (bch, bcl, bu, bv, bnuh, bnul, bnvh, bnvl, bdh, bdl) = best
                vxv = (vxh_s[v], vxl_s[v])
                vyv = (vyh_s[v], vyl_s[v])
                d00v = (d00h_s[v], d00l_s[v])
                d02v = (d02h_s[v], d02l_s[v])
                sv = (sh_s[v], sl_s[v])
                dot01 = _df_add(_df_mul(vxu, vxv), _df_mul(vyu, vyv))
                den = _df_sub(_df_mul(d00u, d00v), _df_sq(dot01))
                nu = _df_sub(_df_mul(d00v, d02u), _df_mul(dot01, d02v))
                nv = _df_sub(_df_mul(d00u, d02v), _df_mul(dot01, d02u))
                nsum = _df_add(nu, nv)
                valid = (_df_pos(den) & _df_nonneg(nu) & _df_nonneg(nv)
                         & _df_le(nsum, den) & u_ok & (v != cidx))
                cost = _df_add(su, sv)
                take = valid & _df_lt(cost, (bch, bcl))
                return (jnp.where(take, cost[0], bch),
                        jnp.where(take, cost[1], bcl),
                        jnp.where(take, u, bu),
                        jnp.where(take, v, bv),
                        jnp.where(take, nu[0], bnuh),
                        jnp.where(take, nu[1], bnul),
                        jnp.where(take, nv[0], bnvh),
                        jnp.where(take, nv[1], bnvl),
                        jnp.where(take, den[0], bdh),
                        jnp.where(take, den[1], bdl))

            return lax.fori_loop(u + jnp.int32(1), jnp.int32(N_NEIGH), pair_v, best)

        best0 = (inf, fz, iz, iz, fz, fz, fz, fz,
                 jnp.ones((VPW,), F32), fz)
        (bch, _, bu, bv, bnuh, bnul, bnvh, bnvl, bdh, bdl) = (
            lax.fori_loop(np.int32(0), np.int32(N_NEIGH), pair_u, best0))

        novalid = bch == inf
        dsum = bdh + bdl
        dsum = jnp.where(novalid, jnp.ones((VPW,), F32), dsum)
        wu = (bnuh + bnul) / dsum
        wv = (bnvh + bnvl) / dsum
        w0 = F32(1.0) - wu - wv
        d2u = (plsc.load_gather(d2h_s, [bu, lanes]),
               plsc.load_gather(d2l_s, [bu, lanes]))
        d2v = (plsc.load_gather(d2h_s, [bv, lanes]),
               plsc.load_gather(d2l_s, [bv, lanes]))
        swap = _df_lt(d2v, d2u)
        store_rows(j, novalid, cidx, bu, bv, swap, w0, wu, wv)

    def site(j, _):
        tx = tx_v[j]
        ty = ty_v[j]

        def fdist_k(k, carry):
            mf, cidx, amb = carry
            dx = tx - px_v[k]
            dy = ty - py_v[k]
            d2 = dx * dx + dy * dy
            d2h_s[k] = d2
            sh_s[k] = d2 * _rsqrt_f32(d2)
            amb = amb | (jnp.abs(d2 - mf) <= _MCMP * (d2 + mf))
            less = d2 < mf
            return (jnp.where(less, d2, mf),
                    jnp.where(less, k, cidx),
                    amb)

        _, cidxf, amb = plsc.parallel_loop(
            np.int32(0), np.int32(N_NEIGH),
            carry=(bigv, iz, fals))(lambda k, c: fdist_k(k, c))

        pcx = plsc.load_gather(px_v, [cidxf, lanes])
        pcy = plsc.load_gather(py_v, [cidxf, lanes])
        v2xf = tx - pcx
        v2yf = ty - pcy

        def fdots_k(k, _):
            vx = px_v[k] - pcx
            vy = py_v[k] - pcy
            vxh_s[k] = vx
            vyh_s[k] = vy
            m1 = vx * v2yf
            m2 = vy * v2xf
            d00h_s[k] = m1 - m2
            d02h_s[k] = _MREL * (jnp.abs(m1) + jnp.abs(m2))
            return jnp.int32(0)

        plsc.parallel_loop(
            np.int32(0), np.int32(N_NEIGH),
            carry=jnp.int32(0))(lambda k, c: fdots_k(k, c))

        def fpair_u(u, best):
            vxu = vxh_s[u]
            vyu = vyh_s[u]
            cu = d00h_s[u]
            mcbu = d02h_s[u]
            nmcbu = -mcbu
            su = sh_s[u]
            u_ok = u != cidxf

            def fpair_v(v, best):
                bc, bu, bv, amb, mac = best
                vxv = vxh_s[v]
                vyv = vyh_s[v]
                cv = d00h_s[v]
                mcbv = d02h_s[v]
                sv = sh_s[v]
                m1 = vxu * vyv
                m2 = vyu * vxv
                cc = m1 - m2
                mcc = _MREL * (jnp.abs(m1) + jnp.abs(m2))
                cpos = cc > fz
                x = jnp.where(cpos, -cv, cv)
                y = jnp.where(cpos, cu, -cu)
                a = cu - cv
                z = jnp.where(cpos, a, -a)
                absc = jnp.abs(cc)
                r = absc - z
                mz = mcc + mcbu + mcbv
                pmask = u_ok & (v != cidxf)
                dvalid = ((absc > mcc) & (x > mcbv)
                          & (y > mcbu) & (r > mz) & pmask)
                dinv = ((x < -mcbv) | (y < nmcbu)
                        | (r < -mz) | ~pmask)
                pamb = ~dvalid & ~dinv
                cost = su + sv
                mc = _MCMP * (cost + bc)
                amb = amb | (dvalid & (jnp.abs(cost - bc) <= mc))
                mac = jnp.where(pamb & (cost < mac), cost, mac)
                take = dvalid & (cost < bc)
                return (jnp.where(take, cost, bc),
                        jnp.where(take, u, bu),
                        jnp.where(take, v, bv),
                        amb, mac)

            return plsc.parallel_loop(
                np.int32(u + 1), np.int32(N_NEIGH),
                unroll=2, carry=best)(lambda v, b: fpair_v(v, b))

        best = (bigv, iz, iz, amb, bigv)
        for _u in range(N_NEIGH - 1):
            best = fpair_u(_u, best)
        bc, bu, bv, amb, mac = best

        amb = amb | ((mac < bigv) & (mac < bc + _MCMP * (bc + mac)))
        novalid = bc == bigv
        d2uf = plsc.load_gather(d2h_s, [bu, lanes])
        d2vf = plsc.load_gather(d2h_s, [bv, lanes])
        amb = amb | (~novalid & (jnp.abs(d2vf - d2uf)
                                 <= _MCMP * (d2uf + d2vf)))
        swap = d2vf < d2uf

        pxu = plsc.load_gather(px_v, [bu, lanes])
        pyu = plsc.load_gather(py_v, [bu, lanes])
        pxv = plsc.load_gather(px_v, [bv, lanes])
        pyv = plsc.load_gather(py_v, [bv, lanes])
        vxu = _df_diff(pxu, pcx)
        vyu = _df_diff(pyu, pcy)
        vxv = _df_diff(pxv, pcx)
        vyv = _df_diff(pyv, pcy)
        v2x = _df_diff(tx, pcx)
        v2y = _df_diff(ty, pcy)
        ccd = _df_sub(_df_mul(vxu, vyv), _df_mul(vyu, vxv))
        cud = _df_sub(_df_mul(vxu, v2y), _df_mul(vyu, v2x))
        cvd = _df_sub(_df_mul(vxv, v2y), _df_mul(vyv, v2x))
        dh = ccd[0] + ccd[1]
        dh = jnp.where(novalid | (dh == fz), jnp.ones((VPW,), F32), dh)
        wu = -(cvd[0] + cvd[1]) / dh
        wv = (cud[0] + cud[1]) / dh
        w0 = F32(1.0) - wu - wv
        store_rows(j, novalid, cidxf, bu, bv, swap, w0, wu, wv)

        any_amb = jnp.max(jnp.where(amb, jnp.ones((VPW,), F32), fz)) > F32(0.0)

        @pl.when(any_amb)
        def _():
            site_slow(j, tx, ty)

        return jnp.int32(0)

    lax.fori_loop(np.int32(0), np.int32(N_SITES), site, jnp.int32(0))
    pltpu.sync_copy(ow_v, w_hbm.at[wid])
    pltpu.sync_copy(oi_v, i_hbm.at[wid])


_bary_sc = functools.partial(
    pl.kernel,
    out_type=[
        jax.ShapeDtypeStruct((NW, N_SITES * 3, VPW), F32),
        jax.ShapeDtypeStruct((NW, N_SITES * 3, VPW), I32),
    ],
    mesh=plsc.VectorSubcoreMesh(
        core_axis_name="c", subcore_axis_name="s",
        num_cores=2, num_subcores=16),
    compiler_params=pltpu.CompilerParams(needs_layout_passes=False),
    scratch_types=(
        [pltpu.VMEM((N_NEIGH, VPW), F32)] * 2
        + [pltpu.VMEM((N_SITES, VPW), F32)] * 2
        + [pltpu.VMEM((N_NEIGH, VPW), F32)] * 12
        + [pltpu.VMEM((N_SITES * 3, VPW), F32),
           pltpu.VMEM((N_SITES * 3, VPW), I32)]
    ),
)(_bary_body)


def kernel(template, projections):
    t = template.astype(F32).reshape(N_SITES, 2)
    p = projections.astype(F32)
    px = p[..., 0].reshape(NW, VPW, N_NEIGH).transpose(0, 2, 1)
    py = p[..., 1].reshape(NW, VPW, N_NEIGH).transpose(0, 2, 1)
    txb = jnp.broadcast_to(t[:, 0][:, None], (N_SITES, VPW)) + F32(0.0)
    tyb = jnp.broadcast_to(t[:, 1][:, None], (N_SITES, VPW)) + F32(0.0)
    with _enable_x64(False):
        w, idx = _bary_sc(px, py, txb, tyb)
    w = w.reshape(NW, N_SITES, 3, VPW).transpose(0, 3, 1, 2)
    w = w.reshape(N_VERTICES, 5, 8, 3)
    idx = idx.reshape(NW, N_SITES, 3, VPW).transpose(0, 3, 1, 2)
    idx = idx.reshape(N_VERTICES, 5, 8, 3)
    return w.astype(jnp.float64), idx.astype(jnp.int64)
